# baseline (speedup 1.0000x reference)
import jax
import jax.numpy as jnp
from jax.experimental import pallas as pl
from jax.experimental.pallas import tpu as pltpu


_BLOCK = 2048


def _rows_kernel(fb_ref, lbl_ref, out_ref):
    ft = fb_ref[...].T
    idx = jnp.argmax(ft, axis=0).astype(jnp.int32)
    lbl = lbl_ref[0, 0, :]
    out_ref[0, 0, :] = jnp.where(lbl == 0, idx, lbl)


def kernel(rel_logits, freq_bias, rel_labels, rel_covar, gamma):
    n, c = freq_bias.shape
    grid = n // _BLOCK
    lbl3 = rel_labels.reshape(grid, 1, _BLOCK)
    out = pl.pallas_call(
        _rows_kernel,
        grid=(grid,),
        in_specs=[
            pl.BlockSpec((_BLOCK, c), lambda i: (i, 0)),
            pl.BlockSpec((1, 1, _BLOCK), lambda i: (i, 0, 0)),
        ],
        out_specs=pl.BlockSpec((1, 1, _BLOCK), lambda i: (i, 0, 0)),
        out_shape=jax.ShapeDtypeStruct((grid, 1, _BLOCK), jnp.int32),
        compiler_params=pltpu.CompilerParams(
            dimension_semantics=("arbitrary",),
        ),
    )(freq_bias, lbl3)
    return out.reshape(n)


# speedup vs baseline: 1.4112x; 1.4112x over previous
import jax
import jax.numpy as jnp
from jax.experimental import pallas as pl
from jax.experimental.pallas import tpu as pltpu


_BLOCK = 16384


def _rows_kernel(fb_ref, lbl_ref, out_ref):
    ft = fb_ref[...].T
    idx = jnp.argmax(ft, axis=0).astype(jnp.int32)
    lbl = lbl_ref[0, 0, :]
    out_ref[0, 0, :] = jnp.where(lbl == 0, idx, lbl)


def kernel(rel_logits, freq_bias, rel_labels, rel_covar, gamma):
    n, c = freq_bias.shape
    grid = n // _BLOCK
    lbl3 = rel_labels.reshape(grid, 1, _BLOCK)
    out = pl.pallas_call(
        _rows_kernel,
        grid=(grid,),
        in_specs=[
            pl.BlockSpec((_BLOCK, c), lambda i: (i, 0)),
            pl.BlockSpec((1, 1, _BLOCK), lambda i: (i, 0, 0)),
        ],
        out_specs=pl.BlockSpec((1, 1, _BLOCK), lambda i: (i, 0, 0)),
        out_shape=jax.ShapeDtypeStruct((grid, 1, _BLOCK), jnp.int32),
        compiler_params=pltpu.CompilerParams(
            dimension_semantics=("arbitrary",),
        ),
    )(freq_bias, lbl3)
    return out.reshape(n)
